# baseline (speedup 1.0000x reference)
import functools

import jax
import jax.numpy as jnp
from jax import lax
from jax.experimental import pallas as pl
from jax.experimental.pallas import tpu as pltpu
from jax.experimental.pallas import tpu_sc as plsc

STEPS = 1000
EMBED = 128
BATCH = 16384

NC = 2
NS = 16
NW = NC * NS
B_PER_W = BATCH // NW

SLAB = 64
SLAB_LAST = STEPS - SLAB * (NS - 1)


def _body(idx_hbm, table_hbm, out_hbm, idx_v, rows_v, table_sp, gsem, ssem):
    c = lax.axis_index("c")
    s = lax.axis_index("s")
    wid = s * NC + c
    base = wid * B_PER_W
    @pl.when(s < NS - 1)
    def _():
        pltpu.async_copy(
            table_hbm.at[pl.ds(s * SLAB, SLAB)],
            table_sp.at[pl.ds(s * SLAB, SLAB)],
            ssem,
        ).wait()

    @pl.when(s == NS - 1)
    def _():
        pltpu.async_copy(
            table_hbm.at[pl.ds((NS - 1) * SLAB, SLAB_LAST)],
            table_sp.at[pl.ds((NS - 1) * SLAB, SLAB_LAST)],
            ssem,
        ).wait()

    pltpu.sync_copy(idx_hbm.at[pl.ds(base, B_PER_W)], idx_v)
    plsc.subcore_barrier()
    pltpu.async_copy(table_sp.at[idx_v], rows_v, gsem).wait()
    pltpu.sync_copy(rows_v, out_hbm.at[pl.ds(base, B_PER_W)])


@functools.partial(
    pl.kernel,
    mesh=plsc.VectorSubcoreMesh(core_axis_name="c", subcore_axis_name="s"),
    out_type=jax.ShapeDtypeStruct((BATCH, EMBED), jnp.float32),
    scratch_types=[
        pltpu.VMEM((B_PER_W,), jnp.int32),
        pltpu.VMEM((B_PER_W, EMBED), jnp.float32),
        pltpu.VMEM_SHARED((STEPS, EMBED), jnp.float32),
        pltpu.SemaphoreType.DMA,
        pltpu.SemaphoreType.DMA,
    ],
)
def _sc_gather(idx_hbm, table_hbm, out_hbm, idx_v, rows_v, table_sp, gsem, ssem):
    _body(idx_hbm, table_hbm, out_hbm, idx_v, rows_v, table_sp, gsem, ssem)


def kernel(x, te_weight):
    idx = x.astype(jnp.int32)
    out = _sc_gather(idx, te_weight)
    return out[:, None, :]


# speedup vs baseline: 1.0347x; 1.0347x over previous
import functools

import jax
import jax.numpy as jnp
from jax import lax
from jax.experimental import pallas as pl
from jax.experimental.pallas import tpu as pltpu
from jax.experimental.pallas import tpu_sc as plsc

STEPS = 1000
EMBED = 128
BATCH = 16384

NC = 2
NS = 16
NW = NC * NS
B_PER_W = BATCH // NW

SLAB = 64
SLAB_LAST = STEPS - SLAB * (NS - 1)

CHUNK = 128
NCHUNK = B_PER_W // CHUNK


def _body(idx_hbm, table_hbm, out_hbm, idx_v, rows_v, table_sp, gsem, ssem, wsem):
    c = lax.axis_index("c")
    s = lax.axis_index("s")
    wid = s * NC + c
    base = wid * B_PER_W
    @pl.when(s < NS - 1)
    def _():
        pltpu.async_copy(
            table_hbm.at[pl.ds(s * SLAB, SLAB)],
            table_sp.at[pl.ds(s * SLAB, SLAB)],
            ssem,
        ).wait()

    @pl.when(s == NS - 1)
    def _():
        pltpu.async_copy(
            table_hbm.at[pl.ds((NS - 1) * SLAB, SLAB_LAST)],
            table_sp.at[pl.ds((NS - 1) * SLAB, SLAB_LAST)],
            ssem,
        ).wait()

    pltpu.sync_copy(idx_hbm.at[pl.ds(base, B_PER_W)], idx_v)
    plsc.subcore_barrier()
    gathers = []
    for j in range(NCHUNK):
        gathers.append(
            pltpu.async_copy(
                table_sp.at[idx_v.at[pl.ds(j * CHUNK, CHUNK)]],
                rows_v.at[pl.ds(j * CHUNK, CHUNK)],
                gsem,
            )
        )
    writes = []
    for j in range(NCHUNK):
        gathers[j].wait()
        writes.append(
            pltpu.async_copy(
                rows_v.at[pl.ds(j * CHUNK, CHUNK)],
                out_hbm.at[pl.ds(base + j * CHUNK, CHUNK)],
                wsem,
            )
        )
    for w in writes:
        w.wait()


@functools.partial(
    pl.kernel,
    mesh=plsc.VectorSubcoreMesh(core_axis_name="c", subcore_axis_name="s"),
    out_type=jax.ShapeDtypeStruct((BATCH, EMBED), jnp.float32),
    scratch_types=[
        pltpu.VMEM((B_PER_W,), jnp.int32),
        pltpu.VMEM((B_PER_W, EMBED), jnp.float32),
        pltpu.VMEM_SHARED((STEPS, EMBED), jnp.float32),
        pltpu.SemaphoreType.DMA,
        pltpu.SemaphoreType.DMA,
        pltpu.SemaphoreType.DMA,
    ],
)
def _sc_gather(idx_hbm, table_hbm, out_hbm, idx_v, rows_v, table_sp, gsem, ssem, wsem):
    _body(idx_hbm, table_hbm, out_hbm, idx_v, rows_v, table_sp, gsem, ssem, wsem)


def kernel(x, te_weight):
    idx = x.astype(jnp.int32)
    out = _sc_gather(idx, te_weight)
    return out[:, None, :]


# speedup vs baseline: 1.0571x; 1.0217x over previous
import functools

import jax
import jax.numpy as jnp
from jax import lax
from jax.experimental import pallas as pl
from jax.experimental.pallas import tpu as pltpu
from jax.experimental.pallas import tpu_sc as plsc

STEPS = 1000
EMBED = 128
BATCH = 16384

NC = 2
NS = 16
NW = NC * NS
B_PER_W = BATCH // NW

SLAB = 64

CHUNK = 64
NCHUNK = B_PER_W // CHUNK


def _body(idx_hbm, table_hbm, out_hbm, idx_v, rows_v, table_sp, gsem, ssem, wsem):
    c = lax.axis_index("c")
    s = lax.axis_index("s")
    wid = s * NC + c
    base = wid * B_PER_W
    row0 = pl.multiple_of(jnp.minimum(s * SLAB, STEPS - SLAB), 8)
    slab_copy = pltpu.async_copy(
        table_hbm.at[pl.ds(row0, SLAB)],
        table_sp.at[pl.ds(row0, SLAB)],
        ssem,
    )
    pltpu.sync_copy(idx_hbm.at[pl.ds(base, B_PER_W)], idx_v)
    slab_copy.wait()
    plsc.subcore_barrier()
    gathers = []
    for j in range(NCHUNK):
        gathers.append(
            pltpu.async_copy(
                table_sp.at[idx_v.at[pl.ds(j * CHUNK, CHUNK)]],
                rows_v.at[pl.ds(j * CHUNK, CHUNK)],
                gsem,
            )
        )
    writes = []
    for j in range(NCHUNK):
        gathers[j].wait()
        writes.append(
            pltpu.async_copy(
                rows_v.at[pl.ds(j * CHUNK, CHUNK)],
                out_hbm.at[pl.ds(base + j * CHUNK, CHUNK)],
                wsem,
            )
        )
    for w in writes:
        w.wait()


@functools.partial(
    pl.kernel,
    mesh=plsc.VectorSubcoreMesh(core_axis_name="c", subcore_axis_name="s"),
    out_type=jax.ShapeDtypeStruct((BATCH, EMBED), jnp.float32),
    scratch_types=[
        pltpu.VMEM((B_PER_W,), jnp.int32),
        pltpu.VMEM((B_PER_W, EMBED), jnp.float32),
        pltpu.VMEM_SHARED((STEPS, EMBED), jnp.float32),
        pltpu.SemaphoreType.DMA,
        pltpu.SemaphoreType.DMA,
        pltpu.SemaphoreType.DMA,
    ],
)
def _sc_gather(idx_hbm, table_hbm, out_hbm, idx_v, rows_v, table_sp, gsem, ssem, wsem):
    _body(idx_hbm, table_hbm, out_hbm, idx_v, rows_v, table_sp, gsem, ssem, wsem)


def kernel(x, te_weight):
    idx = x.astype(jnp.int32)
    out = _sc_gather(idx, te_weight)
    return out[:, None, :]
